# baseline (speedup 1.0000x reference)
import functools

import jax
import jax.numpy as jnp
from jax import lax
from jax.experimental import pallas as pl
from jax.experimental.pallas import tpu as pltpu
from jax.experimental.pallas import tpu_sc as plsc

N = 10000
E = 320000
D = 128
H = D // 2

NS = 16
EPT = E // NS
CHUNK = 125
NCHUNK = EPT // CHUNK
NBUF = 4
NGROUP = NCHUNK // NBUF
N_PAD = 10240
RPT = N_PAD // NS


def _sc_agg_body(xh_hbm, srcs_hbm, dsts_hbm, z_hbm, out_hbm,
                 src_idx, dst_idx, rows, agg_sh, gsem):
    s = lax.axis_index("s")

    pltpu.sync_copy(srcs_hbm.at[s], src_idx)
    pltpu.sync_copy(dsts_hbm.at[s], dst_idx)

    pltpu.sync_copy(z_hbm, agg_sh.at[pl.ds(s * RPT, RPT)])
    plsc.subcore_barrier()

    for b in range(NBUF):
        pltpu.async_copy(xh_hbm.at[src_idx.at[b]], rows.at[b], gsem)

    @pl.loop(0, NGROUP)
    def _outer(k):
        g = k * NBUF
        for b in range(NBUF):
            i = g + b
            pltpu.make_async_copy(
                xh_hbm.at[src_idx.at[i]], rows.at[b], gsem).wait()
            pltpu.sync_copy(rows.at[b], agg_sh.at[dst_idx.at[i]], add=True)
            @pl.when(i + NBUF < NCHUNK)
            def _():
                pltpu.async_copy(
                    xh_hbm.at[src_idx.at[i + NBUF]], rows.at[b], gsem)

    plsc.subcore_barrier()
    pltpu.sync_copy(agg_sh.at[pl.ds(s * RPT, RPT)],
                    out_hbm.at[pl.ds(s * RPT, RPT)])


def _sc_agg_half(xh, srcs, dsts, z):
    mesh = plsc.VectorSubcoreMesh(
        core_axis_name="c", subcore_axis_name="s", num_cores=1)
    fn = pl.kernel(
        _sc_agg_body,
        out_type=jax.ShapeDtypeStruct((N_PAD, H), jnp.float32),
        mesh=mesh,
        scratch_types=[
            pltpu.VMEM((NCHUNK, CHUNK), jnp.int32),
            pltpu.VMEM((NCHUNK, CHUNK), jnp.int32),
            pltpu.VMEM((NBUF, CHUNK, H), jnp.float32),
            pltpu.VMEM_SHARED((N_PAD, H), jnp.float32),
            pltpu.SemaphoreType.DMA,
        ],
        compiler_params=pltpu.CompilerParams(use_tc_tiling_on_sc=False),
    )
    return fn(xh, srcs, dsts, z)


def _tc_body(x_ref, aggl_ref, aggr_ref, wt_ref, b_ref, gamma_ref, beta_ref,
             out_ref):
    a = x_ref[...] + jnp.concatenate(
        [aggl_ref[:N], aggr_ref[:N]], axis=1)
    h = jnp.dot(a, wt_ref[...], preferred_element_type=jnp.float32)
    h = h + b_ref[...]
    mean = jnp.mean(h, axis=0, keepdims=True)
    var = jnp.mean((h - mean) ** 2, axis=0, keepdims=True)
    out_ref[...] = (gamma_ref[...] * (h - mean) * lax.rsqrt(var + 1e-5)
                    + beta_ref[...])


@jax.jit
def kernel(x, edge_index, W, b, gamma, beta):
    src = edge_index[0]
    dst = edge_index[1]
    srcs = src.reshape(NS, NCHUNK, CHUNK)
    dsts = dst.reshape(NS, NCHUNK, CHUNK)
    z = jnp.zeros((RPT, H), jnp.float32)
    xl = x[:, :H]
    xr = x[:, H:]
    aggl = _sc_agg_half(xl, srcs, dsts, z)
    aggr = _sc_agg_half(xr, srcs, dsts, z)

    wt = W.T
    out = pl.pallas_call(
        _tc_body,
        out_shape=jax.ShapeDtypeStruct((N, D), jnp.float32),
    )(x, aggl, aggr, wt, b.reshape(1, D), gamma.reshape(1, D),
      beta.reshape(1, D))
    return out


# speedup vs baseline: 1.3499x; 1.3499x over previous
import functools

import jax
import jax.numpy as jnp
from jax import lax
from jax.experimental import pallas as pl
from jax.experimental.pallas import tpu as pltpu
from jax.experimental.pallas import tpu_sc as plsc

N = 10000
E = 320000
D = 128
H = D // 2

NC = 2
NS = 16
EPT = E // NS
CHUNK = 125
NCHUNK = EPT // CHUNK
NBUF = 5
SLAG = 2
NGROUP = NCHUNK // NBUF
N_PAD = 10240
RPT = N_PAD // NS

assert NCHUNK % NBUF == 0


def _sc_agg_body(xs_hbm, srcs_hbm, dsts_hbm, z_hbm, out_hbm,
                 src_idx, dst_idx, rows, agg_sh, gsem, ssem):
    c = lax.axis_index("c")
    s = lax.axis_index("s")

    pltpu.sync_copy(srcs_hbm.at[c, s], src_idx)
    pltpu.sync_copy(dsts_hbm.at[s], dst_idx)

    pltpu.sync_copy(z_hbm, agg_sh.at[pl.ds(s * RPT, RPT)])
    plsc.subcore_barrier()

    for b in range(NBUF):
        pltpu.async_copy(xs_hbm.at[src_idx.at[b]], rows.at[b], gsem.at[b])

    @pl.loop(0, NGROUP)
    def _outer(k):
        g = k * NBUF
        for b in range(NBUF):
            i = g + b
            pltpu.make_async_copy(
                xs_hbm.at[src_idx.at[i]], rows.at[b], gsem.at[b]).wait()
            pltpu.async_copy(rows.at[b], agg_sh.at[dst_idx.at[i]],
                             ssem.at[b], add=True)
            b2 = (b - SLAG) % NBUF
            j = i - SLAG + NBUF

            @pl.when(i >= SLAG)
            def _():
                pltpu.make_async_copy(
                    rows.at[b2], agg_sh.at[dst_idx.at[i - SLAG]],
                    ssem.at[b2]).wait()

                @pl.when(j < NCHUNK)
                def _():
                    pltpu.async_copy(
                        xs_hbm.at[src_idx.at[j]], rows.at[b2], gsem.at[b2])

    for t in range(SLAG):
        b2 = (NCHUNK - SLAG + t) % NBUF
        pltpu.make_async_copy(
            rows.at[b2],
            agg_sh.at[dst_idx.at[NCHUNK - SLAG + t]], ssem.at[b2]).wait()

    plsc.subcore_barrier()
    pltpu.sync_copy(agg_sh.at[pl.ds(s * RPT, RPT)],
                    out_hbm.at[c, pl.ds(s * RPT, RPT)])


def _sc_agg(xs, srcs, dsts, z):
    mesh = plsc.VectorSubcoreMesh(core_axis_name="c", subcore_axis_name="s")
    fn = pl.kernel(
        _sc_agg_body,
        out_type=jax.ShapeDtypeStruct((NC, N_PAD, H), jnp.float32),
        mesh=mesh,
        scratch_types=[
            pltpu.VMEM((NCHUNK, CHUNK), jnp.int32),
            pltpu.VMEM((NCHUNK, CHUNK), jnp.int32),
            pltpu.VMEM((NBUF, CHUNK, H), jnp.float32),
            pltpu.VMEM_SHARED((N_PAD, H), jnp.float32),
            pltpu.SemaphoreType.DMA((NBUF,)),
            pltpu.SemaphoreType.DMA((NBUF,)),
        ],
        compiler_params=pltpu.CompilerParams(use_tc_tiling_on_sc=False),
    )
    return fn(xs, srcs, dsts, z)


def _tc_body(x_ref, agg_ref, wt_ref, b_ref, gamma_ref, beta_ref, out_ref):
    a = x_ref[...] + jnp.concatenate(
        [agg_ref[0, :N], agg_ref[1, :N]], axis=1)
    h = jnp.dot(a, wt_ref[...], preferred_element_type=jnp.float32)
    h = h + b_ref[...]
    mean = jnp.mean(h, axis=0, keepdims=True)
    var = jnp.mean((h - mean) ** 2, axis=0, keepdims=True)
    out_ref[...] = (gamma_ref[...] * (h - mean) * lax.rsqrt(var + 1e-5)
                    + beta_ref[...])


@jax.jit
def kernel(x, edge_index, W, b, gamma, beta):
    src = edge_index[0]
    dst = edge_index[1]
    xs = jnp.concatenate([x[:, :H], x[:, H:]], axis=0)
    srcs = jnp.stack([src, src + N]).reshape(NC, NS, NCHUNK, CHUNK)
    dsts = dst.reshape(NS, NCHUNK, CHUNK)
    z = jnp.zeros((RPT, H), jnp.float32)
    agg = _sc_agg(xs, srcs, dsts, z)

    wt = W.T
    out = pl.pallas_call(
        _tc_body,
        out_shape=jax.ShapeDtypeStruct((N, D), jnp.float32),
    )(x, agg, wt, b.reshape(1, D), gamma.reshape(1, D), beta.reshape(1, D))
    return out


# speedup vs baseline: 1.4280x; 1.0578x over previous
import functools

import jax
import jax.numpy as jnp
from jax import lax
from jax.experimental import pallas as pl
from jax.experimental.pallas import tpu as pltpu
from jax.experimental.pallas import tpu_sc as plsc

N = 10000
E = 320000
D = 128
H = D // 2

NC = 2
NS = 16
EPT = E // NS
CHUNK = 125
NCHUNK = EPT // CHUNK
NBUF = 4
NGROUP = NCHUNK // NBUF
N_PAD = 10240
RPT = N_PAD // NS

assert NCHUNK % NBUF == 0


def _sc_agg_body(xs_hbm, srcs_hbm, dsts_hbm, out_hbm,
                 src_idx, dst_idx, rows, agg_sh, gsem):
    c = lax.axis_index("c")
    s = lax.axis_index("s")

    pltpu.sync_copy(srcs_hbm.at[c, s], src_idx)
    pltpu.sync_copy(dsts_hbm.at[s], dst_idx)

    pltpu.sync_copy(xs_hbm.at[pl.ds(c * N_PAD + s * RPT, RPT)],
                    agg_sh.at[pl.ds(s * RPT, RPT)])
    plsc.subcore_barrier()

    for b in range(NBUF):
        pltpu.async_copy(xs_hbm.at[src_idx.at[b]], rows.at[b], gsem.at[b])

    @pl.loop(0, NGROUP)
    def _outer(k):
        g = k * NBUF
        for b in range(NBUF):
            i = g + b
            pltpu.make_async_copy(
                xs_hbm.at[src_idx.at[i]], rows.at[b], gsem.at[b]).wait()
            pltpu.sync_copy(rows.at[b], agg_sh.at[dst_idx.at[i]], add=True)
            @pl.when(i + NBUF < NCHUNK)
            def _():
                pltpu.async_copy(
                    xs_hbm.at[src_idx.at[i + NBUF]], rows.at[b], gsem.at[b])

    plsc.subcore_barrier()
    pltpu.sync_copy(agg_sh.at[pl.ds(s * RPT, RPT)],
                    out_hbm.at[c, pl.ds(s * RPT, RPT)])


def _sc_agg(xs, srcs, dsts):
    mesh = plsc.VectorSubcoreMesh(core_axis_name="c", subcore_axis_name="s")
    fn = pl.kernel(
        _sc_agg_body,
        out_type=jax.ShapeDtypeStruct((NC, N_PAD, H), jnp.float32),
        mesh=mesh,
        scratch_types=[
            pltpu.VMEM((NCHUNK, CHUNK), jnp.int32),
            pltpu.VMEM((NCHUNK, CHUNK), jnp.int32),
            pltpu.VMEM((NBUF, CHUNK, H), jnp.float32),
            pltpu.VMEM_SHARED((N_PAD, H), jnp.float32),
            pltpu.SemaphoreType.DMA((NBUF,)),
        ],
        compiler_params=pltpu.CompilerParams(use_tc_tiling_on_sc=False),
    )
    return fn(xs, srcs, dsts)


def _tc_body(agg_ref, wt_ref, b_ref, gamma_ref, beta_ref, out_ref):
    a = jnp.concatenate([agg_ref[0, :N], agg_ref[1, :N]], axis=1)
    h = jnp.dot(a, wt_ref[...], preferred_element_type=jnp.float32)
    h = h + b_ref[...]
    mean = jnp.mean(h, axis=0, keepdims=True)
    var = jnp.mean((h - mean) ** 2, axis=0, keepdims=True)
    out_ref[...] = (gamma_ref[...] * (h - mean) * lax.rsqrt(var + 1e-5)
                    + beta_ref[...])


@jax.jit
def kernel(x, edge_index, W, b, gamma, beta):
    src = edge_index[0]
    dst = edge_index[1]
    xp = jnp.concatenate([x, jnp.zeros((N_PAD - N, D), jnp.float32)], axis=0)
    xs = jnp.concatenate([xp[:, :H], xp[:, H:]], axis=0)
    srcs = jnp.stack([src, src + N_PAD]).reshape(NC, NS, NCHUNK, CHUNK)
    dsts = dst.reshape(NS, NCHUNK, CHUNK)
    agg = _sc_agg(xs, srcs, dsts)

    wt = W.T
    out = pl.pallas_call(
        _tc_body,
        out_shape=jax.ShapeDtypeStruct((N, D), jnp.float32),
    )(agg, wt, b.reshape(1, D), gamma.reshape(1, D), beta.reshape(1, D))
    return out
